# baseline (speedup 1.0000x reference)
import functools

import jax
import jax.numpy as jnp
from jax import lax
from jax.experimental import pallas as pl
from jax.experimental.pallas import tpu as pltpu
from jax.experimental.pallas import tpu_sc as plsc

_D_EMBED = 64
_SCALE = float(_D_EMBED) ** 0.5
_LANES = 16
_NUM_WORKERS = 32
_NBUF = 2


def _make_lookup(nrow: int, ncol: int):
    assert nrow % (_NUM_WORKERS * _NBUF) == 0
    rpw = nrow // _NUM_WORKERS
    mesh = plsc.VectorSubcoreMesh(core_axis_name="c", subcore_axis_name="s")

    @functools.partial(
        pl.kernel,
        mesh=mesh,
        out_type=jax.ShapeDtypeStruct((nrow, ncol, _D_EMBED), jnp.float32),
        scratch_types=[
            pltpu.VMEM((rpw, ncol), jnp.float32),
            pltpu.VMEM((rpw, ncol), jnp.int32),
            pltpu.VMEM((_NBUF, ncol, _D_EMBED), jnp.float32),
            pltpu.VMEM((_NBUF, ncol, _D_EMBED), jnp.float32),
            [pltpu.SemaphoreType.DMA] * _NBUF,
            [pltpu.SemaphoreType.DMA] * _NBUF,
        ],
        compiler_params=pltpu.CompilerParams(use_tc_tiling_on_sc=False),
    )
    def lookup(table_hbm, idx_hbm, out_hbm, idx_f, idx_v, gbuf, wbuf, gsems, wsems):
        wid = lax.axis_index("s") * 2 + lax.axis_index("c")
        base = wid * rpw
        pltpu.sync_copy(idx_hbm.at[pl.ds(base, rpw)], idx_f)

        @plsc.parallel_loop(0, rpw, unroll=4)
        def _(r):
            for j in range(ncol // _LANES):
                sl = pl.ds(j * _LANES, _LANES)
                idx_v[r, sl] = jax.lax.bitcast_convert_type(idx_f[r, sl], jnp.int32)
            tail = ncol - (ncol // _LANES) * _LANES
            if tail:
                sl = pl.ds(ncol - _LANES, _LANES)
                idx_v[r, sl] = jax.lax.bitcast_convert_type(idx_f[r, sl], jnp.int32)

        def gather_start(row, b):
            pltpu.async_copy(
                table_hbm.at[idx_v.at[row]], gbuf.at[b], gsems[b]
            )

        for b in range(_NBUF):
            gather_start(b, b)

        @pl.loop(0, rpw, step=_NBUF)
        def _(g0):
            for b in range(_NBUF):
                g = g0 + b

                @pl.when(g >= _NBUF)
                def _():
                    pltpu.make_async_copy(
                        wbuf.at[b], out_hbm.at[base], wsems[b]
                    ).wait()

                pltpu.make_async_copy(
                    table_hbm.at[idx_v.at[g]], gbuf.at[b], gsems[b]
                ).wait()

                @plsc.parallel_loop(0, ncol, unroll=8)
                def _(i):
                    for j in range(_D_EMBED // _LANES):
                        sl = pl.ds(j * _LANES, _LANES)
                        wbuf[b, i, sl] = gbuf[b, i, sl] * _SCALE

                pltpu.async_copy(wbuf.at[b], out_hbm.at[base + g], wsems[b])

                @pl.when(g + _NBUF < rpw)
                def _():
                    gather_start(g + _NBUF, b)

        for b in range(_NBUF):
            pltpu.make_async_copy(
                wbuf.at[b], out_hbm.at[base], wsems[b]
            ).wait()

    return lookup


def kernel(inp, emb_weight):
    b, t = inp.shape
    idx_bits = jax.lax.bitcast_convert_type(inp, jnp.float32)
    return _make_lookup(b, t)(emb_weight, idx_bits)


# speedup vs baseline: 1.0032x; 1.0032x over previous
import functools

import jax
import jax.numpy as jnp
from jax import lax
from jax.experimental import pallas as pl
from jax.experimental.pallas import tpu as pltpu
from jax.experimental.pallas import tpu_sc as plsc

_D_EMBED = 64
_SCALE = float(_D_EMBED) ** 0.5
_LANES = 16
_NUM_WORKERS = 32
_NBUF = 2
_SPLIT = 128


def _make_lookup(nrow: int, ncol: int):
    assert nrow % (_NUM_WORKERS * _NBUF) == 0
    rpw = nrow // _NUM_WORKERS
    nca = _SPLIT
    ncb = ncol - _SPLIT
    mesh = plsc.VectorSubcoreMesh(core_axis_name="c", subcore_axis_name="s")

    @functools.partial(
        pl.kernel,
        mesh=mesh,
        out_type=jax.ShapeDtypeStruct((nrow * ncol, _D_EMBED), jnp.float32),
        scratch_types=[
            pltpu.VMEM((rpw, nca), jnp.float32),
            pltpu.VMEM((rpw, ncb), jnp.float32),
            pltpu.VMEM((rpw * ncol,), jnp.int32),
            pltpu.VMEM((_NBUF, ncol, _D_EMBED), jnp.float32),
            pltpu.VMEM((_NBUF, ncol, _D_EMBED), jnp.float32),
            [pltpu.SemaphoreType.DMA] * _NBUF,
            [pltpu.SemaphoreType.DMA] * _NBUF,
        ],
        compiler_params=pltpu.CompilerParams(use_tc_tiling_on_sc=False),
    )
    def lookup(
        table_hbm, idxa_hbm, idxb_hbm, out_hbm,
        idx_a, idx_b, idx_v, gbuf, wbuf, gsems, wsems,
    ):
        wid = lax.axis_index("s") * 2 + lax.axis_index("c")
        base = wid * rpw
        pltpu.sync_copy(idxa_hbm.at[pl.ds(base, rpw)], idx_a)
        pltpu.sync_copy(idxb_hbm.at[pl.ds(base, rpw)], idx_b)

        @plsc.parallel_loop(0, rpw, unroll=4)
        def _(r):
            for j in range(nca // _LANES):
                sl = pl.ds(j * _LANES, _LANES)
                dst = pl.ds(r * ncol + j * _LANES, _LANES)
                idx_v[dst] = jax.lax.bitcast_convert_type(idx_a[r, sl], jnp.int32)
            for j in range(ncb // _LANES):
                sl = pl.ds(j * _LANES, _LANES)
                dst = pl.ds(r * ncol + nca + j * _LANES, _LANES)
                idx_v[dst] = jax.lax.bitcast_convert_type(idx_b[r, sl], jnp.int32)
            tail = ncb - (ncb // _LANES) * _LANES
            if tail:
                sl = pl.ds(ncb - _LANES, _LANES)
                dst = pl.ds(r * ncol + ncol - _LANES, _LANES)
                idx_v[dst] = jax.lax.bitcast_convert_type(idx_b[r, sl], jnp.int32)

        def gather_start(row, b):
            pltpu.async_copy(
                table_hbm.at[idx_v.at[pl.ds(row * ncol, ncol)]],
                gbuf.at[b],
                gsems[b],
            )

        for b in range(_NBUF):
            gather_start(b, b)

        @pl.loop(0, rpw, step=_NBUF)
        def _(g0):
            for b in range(_NBUF):
                g = g0 + b

                @pl.when(g >= _NBUF)
                def _():
                    pltpu.make_async_copy(
                        wbuf.at[b],
                        out_hbm.at[pl.ds(base * ncol, ncol)],
                        wsems[b],
                    ).wait()

                pltpu.make_async_copy(
                    table_hbm.at[idx_v.at[pl.ds(g * ncol, ncol)]],
                    gbuf.at[b],
                    gsems[b],
                ).wait()

                @plsc.parallel_loop(0, ncol, unroll=8)
                def _(i):
                    for j in range(_D_EMBED // _LANES):
                        sl = pl.ds(j * _LANES, _LANES)
                        wbuf[b, i, sl] = gbuf[b, i, sl] * _SCALE

                pltpu.async_copy(
                    wbuf.at[b],
                    out_hbm.at[pl.ds((base + g) * ncol, ncol)],
                    wsems[b],
                )

                @pl.when(g + _NBUF < rpw)
                def _():
                    gather_start(g + _NBUF, b)

        for b in range(_NBUF):
            pltpu.make_async_copy(
                wbuf.at[b], out_hbm.at[pl.ds(base * ncol, ncol)], wsems[b]
            ).wait()

    return lookup


def kernel(inp, emb_weight):
    b, t = inp.shape
    bits = jax.lax.bitcast_convert_type(inp, jnp.float32)
    idx_a = bits[:, :_SPLIT]
    idx_b = bits[:, _SPLIT:]
    flat = _make_lookup(b, t)(emb_weight, idx_a, idx_b)
    return flat.reshape(b, t, _D_EMBED)
